# baseline (speedup 1.0000x reference)
import jax
import jax.numpy as jnp
from jax import lax
from jax.experimental import pallas as pl

_SQRT2 = 1.4142135623730951


def _gelu(x):
    return x * 0.5 * (1.0 + lax.erf(x / _SQRT2))


def _ln(x, g, b):
    m = jnp.mean(x, axis=-1, keepdims=True)
    d = x - m
    v = jnp.mean(d * d, axis=-1, keepdims=True)
    return d * lax.rsqrt(v + 1e-5) * g + b


def _dot(a, b):
    return jnp.dot(a, b, preferred_element_type=jnp.float32)


def _bdot(a, b):
    return jnp.dot(a.astype(jnp.bfloat16), b.astype(jnp.bfloat16),
                   preferred_element_type=jnp.float32)


def _pick_block(e):
    for b in (2000, 1600, 1280, 1024, 1000, 800, 640, 512, 256, 128, 64, 32, 16, 8):
        if e % b == 0:
            return b
    return e


def _qf_kernel(q_ref, qfg_ref, qfb_ref, qfW_ref, qfbias_ref, type2_ref, out_ref):
    q = q_ref[...]
    y = _dot(_ln(q, qfg_ref[...], qfb_ref[...]), qfW_ref[...]) + qfbias_ref[...]
    out_ref[...] = _gelu(y) + type2_ref[...]


def _edge_fwd_kernel(et_ref, eb_ref, ord_ref, mask_ref, qf2_ref, ordemb_ref,
                     m1g_ref, m1b_ref, m1W_ref, m1bias_ref,
                     m2g_ref, m2b_ref, m2W_ref, m2bias_ref,
                     x_ref, ssum_ref, cnt_ref):
    i = pl.program_id(0)
    B = et_ref.shape[0]
    G = qf2_ref.shape[0]
    R = ordemb_ref.shape[0]

    eb = eb_ref[0, 0, :]
    P = (eb[:, None] == lax.broadcasted_iota(jnp.int32, (B, G), 1)).astype(jnp.bfloat16)
    oidx = jnp.clip(ord_ref[0, 0, :], -1, R - 2) + 1
    Po = (oidx[:, None] == lax.broadcasted_iota(jnp.int32, (B, R), 1)).astype(jnp.bfloat16)

    et = et_ref[...] + _bdot(Po, ordemb_ref[...]) + _bdot(P, qf2_ref[...])
    x = _gelu(_bdot(_ln(et, m1g_ref[...], m1b_ref[...]), m1W_ref[...]) + m1bias_ref[...])
    x = _gelu(_bdot(_ln(x, m2g_ref[...], m2b_ref[...]), m2W_ref[...]) + m2bias_ref[...])
    x_ref[...] = x.astype(jnp.bfloat16)

    mf = mask_ref[0, 0, :].astype(jnp.float32)
    xm = (x * mf[:, None]).astype(jnp.bfloat16)
    psum = lax.dot_general(P, xm, (((0,), (0,)), ((), ())),
                           preferred_element_type=jnp.float32)
    pcnt = lax.dot_general(P.astype(jnp.float32), mf[:, None], (((0,), (0,)), ((), ())),
                           preferred_element_type=jnp.float32)

    @pl.when(i == 0)
    def _init():
        ssum_ref[...] = jnp.zeros_like(ssum_ref)
        cnt_ref[...] = jnp.zeros_like(cnt_ref)

    ssum_ref[...] += psum
    cnt_ref[...] += pcnt


def _graph_kernel(ssum_ref, cnt_ref, q_ref, gng_ref, gnb_ref,
                  spg_ref, spb_ref, spW1_ref, spb1_ref, spW2_ref, spb2_ref,
                  cs_ref, stop_ref, tab_ref, r_ref):
    ssum = ssum_ref[...]
    q = q_ref[...]
    cnt_raw = cnt_ref[...]
    cnt_c = jnp.maximum(cnt_raw, 1.0)
    cs = _ln(ssum / cnt_c + q, gng_ref[...], gnb_ref[...])
    cs_ref[...] = cs
    r = 1.0 / (cnt_raw + 1.0)
    r_ref[...] = r
    tab_ref[...] = jnp.concatenate([cs, ssum * r + q], axis=1)
    catg = jnp.concatenate([cs, q], axis=1)
    hs = _gelu(_dot(_ln(catg, spg_ref[...], spb_ref[...]), spW1_ref[...]) + spb1_ref[...])
    stop_ref[...] = _dot(hs, spW2_ref[...]) + spb2_ref[...]


def _edge_head_kernel(x_ref, eb_ref, tab_ref, r_ref, gng_ref, gnb_ref,
                      lhg_ref, lhb_ref, lhW1_ref, lhb1_ref, lhW2_ref, lhb2_ref,
                      out_ref):
    B = x_ref.shape[0]
    G = tab_ref.shape[0]
    H = x_ref.shape[1]

    eb = eb_ref[0, 0, :]
    P = (eb[:, None] == lax.broadcasted_iota(jnp.int32, (B, G), 1)).astype(jnp.bfloat16)
    gath = _bdot(P, tab_ref[...])
    cs_e = gath[:, :H]
    a_e = gath[:, H:]
    r_e = _bdot(P, r_ref[...])
    v = x_ref[...].astype(jnp.float32) * r_e + a_e
    ns = _ln(v, gng_ref[...], gnb_ref[...])
    cat = jnp.concatenate([cs_e, ns], axis=1)
    h = _gelu(_bdot(_ln(cat, lhg_ref[...], lhb_ref[...]), lhW1_ref[...]) + lhb1_ref[...])
    out_ref[...] = _bdot(h, lhW2_ref[...]) + lhb2_ref[...]


def kernel(edge_tokens, question_tokens, edge_batch, selected_mask, selection_order,
           order_emb, type_emb, gn_g, gn_b, qf_g, qf_b, qf_W, qf_bias,
           m1_g, m1_b, m1_W, m1_bias, m2_g, m2_b, m2_W, m2_bias,
           lh_g, lh_b, lh_W1, lh_b1, lh_W2, lh_b2,
           sp_g, sp_b, sp_W1, sp_b1, sp_W2, sp_b2):
    E, H = edge_tokens.shape
    G = question_tokens.shape[0]
    B = _pick_block(E)
    NB = E // B

    row = lambda a: a.reshape(1, -1)
    ordemb = jnp.concatenate(
        [order_emb, jnp.zeros((16 - order_emb.shape[0], H), order_emb.dtype)], axis=0)

    qf2 = pl.pallas_call(
        _qf_kernel,
        out_shape=jax.ShapeDtypeStruct((G, H), jnp.float32),
    )(question_tokens, row(qf_g), row(qf_b), qf_W, row(qf_bias), row(type_emb[2]))

    eb3 = edge_batch.astype(jnp.int32).reshape(NB, 1, B)
    ord3 = selection_order.astype(jnp.int32).reshape(NB, 1, B)
    mask3 = selected_mask.astype(jnp.int32).reshape(NB, 1, B)

    full = lambda shape: pl.BlockSpec(shape, lambda i: (0,) * len(shape))
    idx3 = pl.BlockSpec((1, 1, B), lambda i: (i, 0, 0))
    eblk = pl.BlockSpec((B, H), lambda i: (i, 0))

    x, ssum, cnt = pl.pallas_call(
        _edge_fwd_kernel,
        grid=(NB,),
        in_specs=[eblk, idx3, idx3, idx3, full((G, H)), full((16, H)),
                  full((1, H)), full((1, H)), full((H, H)), full((1, H)),
                  full((1, H)), full((1, H)), full((H, H)), full((1, H))],
        out_specs=[eblk, full((G, H)), full((G, 1))],
        out_shape=[jax.ShapeDtypeStruct((E, H), jnp.bfloat16),
                   jax.ShapeDtypeStruct((G, H), jnp.float32),
                   jax.ShapeDtypeStruct((G, 1), jnp.float32)],
    )(edge_tokens, eb3, ord3, mask3, qf2, ordemb,
      row(m1_g), row(m1_b), m1_W, row(m1_bias),
      row(m2_g), row(m2_b), m2_W, row(m2_bias))

    cs, stop, tab, r = pl.pallas_call(
        _graph_kernel,
        out_shape=[jax.ShapeDtypeStruct((G, H), jnp.float32),
                   jax.ShapeDtypeStruct((G, 1), jnp.float32),
                   jax.ShapeDtypeStruct((G, 2 * H), jnp.float32),
                   jax.ShapeDtypeStruct((G, 1), jnp.float32)],
    )(ssum, cnt, question_tokens, row(gn_g), row(gn_b),
      row(sp_g), row(sp_b), sp_W1, row(sp_b1), sp_W2, sp_b2.reshape(1, 1))

    logits = pl.pallas_call(
        _edge_head_kernel,
        grid=(NB,),
        in_specs=[eblk, idx3, full((G, 2 * H)), full((G, 1)),
                  full((1, H)), full((1, H)),
                  full((1, 2 * H)), full((1, 2 * H)), full((2 * H, H)),
                  full((1, H)), full((H, 1)), full((1, 1))],
        out_specs=pl.BlockSpec((B, 1), lambda i: (i, 0)),
        out_shape=jax.ShapeDtypeStruct((E, 1), jnp.float32),
    )(x, eb3, tab, r, row(gn_g), row(gn_b),
      row(lh_g), row(lh_b), lh_W1, row(lh_b1), lh_W2, lh_b2.reshape(1, 1))

    return logits[:, 0], stop[:, 0], cs


# speedup vs baseline: 1.6032x; 1.6032x over previous
import jax
import jax.numpy as jnp
from jax import lax
from jax.experimental import pallas as pl
from jax.experimental.pallas import tpu as pltpu

_SQRT2 = 1.4142135623730951
_W = 128


def _gelu(x):
    return x * 0.5 * (1.0 + lax.erf(x / _SQRT2))


def _ln(x, g, b):
    m = jnp.mean(x, axis=-1, keepdims=True)
    d = x - m
    v = jnp.mean(d * d, axis=-1, keepdims=True)
    return d * lax.rsqrt(v + 1e-5) * g + b


def _dot(a, b):
    return jnp.dot(a, b, preferred_element_type=jnp.float32)


def _pick_block(e):
    for b in (2000, 1600, 1280, 1024, 1000, 800, 640, 512, 256, 128, 64, 32, 16, 8):
        if e % b == 0:
            return b
    return e


def _qf_kernel(q_ref, qfg_ref, qfb_ref, qfW_ref, qfbias_ref, type2_ref, out_ref):
    q = q_ref[...]
    y = _dot(_ln(q, qfg_ref[...], qfb_ref[...]), qfW_ref[...]) + qfbias_ref[...]
    out_ref[...] = _gelu(y) + type2_ref[...]


def _edge_fwd_kernel(lo_ref, hi_ref, et_ref, eb_ref, ord_ref, mask_ref,
                     qf2_ref, ordemb_ref,
                     m1g_ref, m1b_ref, m1W_ref, m1bias_ref,
                     m2g_ref, m2b_ref, m2W_ref, m2bias_ref,
                     x_ref, ssum_ref, cnt_ref, gbuf_ref):
    i = pl.program_id(0)
    B = et_ref.shape[0]
    G = ssum_ref.shape[0] - _W
    R = ordemb_ref.shape[0]

    lo = (lo_ref[i] // 8) * 8
    narrow = hi_ref[i] - lo < _W

    eb = eb_ref[0, 0, :]

    @pl.when(narrow)
    def _():
        Pw = (eb[:, None] == lax.broadcasted_iota(jnp.int32, (B, _W), 1) + lo
              ).astype(jnp.float32)
        gbuf_ref[...] = _dot(Pw, qf2_ref[pl.ds(lo, _W), :])

    @pl.when(jnp.logical_not(narrow))
    def _():
        P = (eb[:, None] == lax.broadcasted_iota(jnp.int32, (B, G), 1)
             ).astype(jnp.float32)
        gbuf_ref[...] = _dot(P, qf2_ref[:G, :])

    oidx = jnp.clip(ord_ref[0, 0, :], -1, R - 2) + 1
    Po = (oidx[:, None] == lax.broadcasted_iota(jnp.int32, (B, R), 1)
          ).astype(jnp.float32)

    et = et_ref[...] + _dot(Po, ordemb_ref[...]) + gbuf_ref[...]
    x = _gelu(_dot(_ln(et, m1g_ref[...], m1b_ref[...]), m1W_ref[...]) + m1bias_ref[...])
    x = _gelu(_dot(_ln(x, m2g_ref[...], m2b_ref[...]), m2W_ref[...]) + m2bias_ref[...])
    x_ref[...] = x

    mf = mask_ref[0, 0, :].astype(jnp.float32)
    xm = x * mf[:, None]

    @pl.when(i == 0)
    def _():
        ssum_ref[...] = jnp.zeros_like(ssum_ref)
        cnt_ref[...] = jnp.zeros_like(cnt_ref)

    @pl.when(narrow)
    def _():
        Pw = (eb[:, None] == lax.broadcasted_iota(jnp.int32, (B, _W), 1) + lo
              ).astype(jnp.float32)
        ssum_ref[pl.ds(lo, _W), :] += lax.dot_general(
            Pw, xm, (((0,), (0,)), ((), ())), preferred_element_type=jnp.float32)
        cnt_ref[pl.ds(lo, _W), :] += lax.dot_general(
            Pw, mf[:, None], (((0,), (0,)), ((), ())),
            preferred_element_type=jnp.float32)

    @pl.when(jnp.logical_not(narrow))
    def _():
        P = (eb[:, None] == lax.broadcasted_iota(jnp.int32, (B, G), 1)
             ).astype(jnp.float32)
        ssum_ref[:G, :] += lax.dot_general(
            P, xm, (((0,), (0,)), ((), ())), preferred_element_type=jnp.float32)
        cnt_ref[:G, :] += lax.dot_general(
            P, mf[:, None], (((0,), (0,)), ((), ())),
            preferred_element_type=jnp.float32)


def _graph_kernel(ssum_ref, cnt_ref, q_ref, gng_ref, gnb_ref,
                  spg_ref, spb_ref, spW1_ref, spb1_ref, spW2_ref, spb2_ref,
                  cs_ref, stop_ref, tab_ref, r_ref):
    G = q_ref.shape[0]
    ssum = ssum_ref[:G, :]
    q = q_ref[...]
    cnt_raw = cnt_ref[:G, :]
    cnt_c = jnp.maximum(cnt_raw, 1.0)
    cs = _ln(ssum / cnt_c + q, gng_ref[...], gnb_ref[...])
    cs_ref[...] = cs
    r = 1.0 / (cnt_raw + 1.0)
    r_ref[:G, :] = r
    r_ref[G:, :] = jnp.zeros_like(r_ref[G:, :])
    tab_ref[:G, :] = jnp.concatenate([cs, ssum * r + q], axis=1)
    tab_ref[G:, :] = jnp.zeros_like(tab_ref[G:, :])
    catg = jnp.concatenate([cs, q], axis=1)
    hs = _gelu(_dot(_ln(catg, spg_ref[...], spb_ref[...]), spW1_ref[...]) + spb1_ref[...])
    stop_ref[...] = _dot(hs, spW2_ref[...]) + spb2_ref[...]


def _edge_head_kernel(lo_ref, hi_ref, x_ref, eb_ref, tab_ref, r_ref,
                      gng_ref, gnb_ref,
                      lhg_ref, lhb_ref, lhW1_ref, lhb1_ref, lhW2_ref, lhb2_ref,
                      out_ref, gbuf_ref, rbuf_ref):
    i = pl.program_id(0)
    B = x_ref.shape[0]
    G = tab_ref.shape[0] - _W
    H = x_ref.shape[1]

    lo = (lo_ref[i] // 8) * 8
    narrow = hi_ref[i] - lo < _W
    eb = eb_ref[0, 0, :]

    @pl.when(narrow)
    def _():
        Pw = (eb[:, None] == lax.broadcasted_iota(jnp.int32, (B, _W), 1) + lo
              ).astype(jnp.float32)
        gbuf_ref[...] = _dot(Pw, tab_ref[pl.ds(lo, _W), :])
        rbuf_ref[...] = _dot(Pw, r_ref[pl.ds(lo, _W), :])

    @pl.when(jnp.logical_not(narrow))
    def _():
        P = (eb[:, None] == lax.broadcasted_iota(jnp.int32, (B, G), 1)
             ).astype(jnp.float32)
        gbuf_ref[...] = _dot(P, tab_ref[:G, :])
        rbuf_ref[...] = _dot(P, r_ref[:G, :])

    gath = gbuf_ref[...]
    cs_e = gath[:, :H]
    a_e = gath[:, H:]
    v = x_ref[...] * rbuf_ref[...] + a_e
    ns = _ln(v, gng_ref[...], gnb_ref[...])
    cat = jnp.concatenate([cs_e, ns], axis=1)
    h = _gelu(_dot(_ln(cat, lhg_ref[...], lhb_ref[...]), lhW1_ref[...]) + lhb1_ref[...])
    out_ref[...] = _dot(h, lhW2_ref[...]) + lhb2_ref[...]


def kernel(edge_tokens, question_tokens, edge_batch, selected_mask, selection_order,
           order_emb, type_emb, gn_g, gn_b, qf_g, qf_b, qf_W, qf_bias,
           m1_g, m1_b, m1_W, m1_bias, m2_g, m2_b, m2_W, m2_bias,
           lh_g, lh_b, lh_W1, lh_b1, lh_W2, lh_b2,
           sp_g, sp_b, sp_W1, sp_b1, sp_W2, sp_b2):
    E, H = edge_tokens.shape
    G = question_tokens.shape[0]
    B = _pick_block(E)
    NB = E // B
    GP = G + _W

    row = lambda a: a.reshape(1, -1)
    ordemb = jnp.concatenate(
        [order_emb, jnp.zeros((16 - order_emb.shape[0], H), order_emb.dtype)], axis=0)

    qf2 = pl.pallas_call(
        _qf_kernel,
        out_shape=jax.ShapeDtypeStruct((G, H), jnp.float32),
    )(question_tokens, row(qf_g), row(qf_b), qf_W, row(qf_bias), row(type_emb[2]))
    qf2p = jnp.concatenate([qf2, jnp.zeros((_W, H), jnp.float32)], axis=0)

    ebi = edge_batch.astype(jnp.int32)
    los = ebi[0::B]
    his = ebi[B - 1::B]
    eb3 = ebi.reshape(NB, 1, B)
    ord3 = selection_order.astype(jnp.int32).reshape(NB, 1, B)
    mask3 = selected_mask.astype(jnp.int32).reshape(NB, 1, B)

    full = lambda shape: pl.BlockSpec(shape, lambda i: (0,) * len(shape))
    smem = pl.BlockSpec(memory_space=pltpu.SMEM)
    idx3 = pl.BlockSpec((1, 1, B), lambda i: (i, 0, 0))
    eblk = pl.BlockSpec((B, H), lambda i: (i, 0))

    x, ssum, cnt = pl.pallas_call(
        _edge_fwd_kernel,
        grid=(NB,),
        in_specs=[smem, smem, eblk, idx3, idx3, idx3, full((GP, H)), full((16, H)),
                  full((1, H)), full((1, H)), full((H, H)), full((1, H)),
                  full((1, H)), full((1, H)), full((H, H)), full((1, H))],
        out_specs=[eblk, full((GP, H)), full((GP, 1))],
        out_shape=[jax.ShapeDtypeStruct((E, H), jnp.float32),
                   jax.ShapeDtypeStruct((GP, H), jnp.float32),
                   jax.ShapeDtypeStruct((GP, 1), jnp.float32)],
        scratch_shapes=[pltpu.VMEM((B, H), jnp.float32)],
    )(los, his, edge_tokens, eb3, ord3, mask3, qf2p, ordemb,
      row(m1_g), row(m1_b), m1_W, row(m1_bias),
      row(m2_g), row(m2_b), m2_W, row(m2_bias))

    cs, stop, tab, r = pl.pallas_call(
        _graph_kernel,
        out_shape=[jax.ShapeDtypeStruct((G, H), jnp.float32),
                   jax.ShapeDtypeStruct((G, 1), jnp.float32),
                   jax.ShapeDtypeStruct((GP, 2 * H), jnp.float32),
                   jax.ShapeDtypeStruct((GP, 1), jnp.float32)],
    )(ssum, cnt, question_tokens, row(gn_g), row(gn_b),
      row(sp_g), row(sp_b), sp_W1, row(sp_b1), sp_W2, sp_b2.reshape(1, 1))

    logits = pl.pallas_call(
        _edge_head_kernel,
        grid=(NB,),
        in_specs=[smem, smem, eblk, idx3, full((GP, 2 * H)), full((GP, 1)),
                  full((1, H)), full((1, H)),
                  full((1, 2 * H)), full((1, 2 * H)), full((2 * H, H)),
                  full((1, H)), full((H, 1)), full((1, 1))],
        out_specs=pl.BlockSpec((B, 1), lambda i: (i, 0)),
        out_shape=jax.ShapeDtypeStruct((E, 1), jnp.float32),
        scratch_shapes=[pltpu.VMEM((B, 2 * H), jnp.float32),
                        pltpu.VMEM((B, 1), jnp.float32)],
    )(los, his, x, eb3, tab, r, row(gn_g), row(gn_b),
      row(lh_g), row(lh_b), lh_W1, row(lh_b1), lh_W2, lh_b2.reshape(1, 1))

    return logits[:, 0], stop[:, 0], cs


# speedup vs baseline: 1.7722x; 1.1054x over previous
import jax
import jax.numpy as jnp
from jax import lax
from jax.experimental import pallas as pl
from jax.experimental.pallas import tpu as pltpu

_SQRT2 = 1.4142135623730951
_W = 128


def _gelu(x):
    return x * 0.5 * (1.0 + lax.erf(x / _SQRT2))


def _lnn(x):
    m = jnp.mean(x, axis=-1, keepdims=True)
    ms = jnp.mean(x * x, axis=-1, keepdims=True)
    rs = lax.rsqrt(jnp.maximum(ms - m * m, 0.0) + 1e-5)
    return x * rs - m * rs


def _ln(x, g, b):
    m = jnp.mean(x, axis=-1, keepdims=True)
    ms = jnp.mean(x * x, axis=-1, keepdims=True)
    rs = lax.rsqrt(jnp.maximum(ms - m * m, 0.0) + 1e-5)
    return (x * rs - m * rs) * g + b


def _dot(a, b):
    return jnp.dot(a, b, preferred_element_type=jnp.float32)


def _pick_block(e):
    for b in (4000, 2000, 1600, 1280, 1024, 1000, 800, 640, 512, 256, 128, 64, 32, 16, 8):
        if e % b == 0:
            return b
    return e


def _qf_kernel(q_ref, qfW_ref, qfbias_ref, out_ref):
    y = _dot(_lnn(q_ref[...]), qfW_ref[...]) + qfbias_ref[...]
    out_ref[...] = _gelu(y)


def _edge_fwd_kernel(lo_ref, hi_ref, et_ref, eb_ref, ord_ref, mask_ref,
                     qf2_ref, ordemb_ref, m1W_ref, m1bias_ref, m2W_ref, m2bias_ref,
                     x_ref, ssum_ref, cnt_ref, gbuf_ref, pw_ref):
    i = pl.program_id(0)
    B = et_ref.shape[0]
    G = ssum_ref.shape[0] - _W
    R = ordemb_ref.shape[0]

    lo = (lo_ref[i] // 8) * 8
    narrow = hi_ref[i] - lo < _W

    eb = eb_ref[0, 0, :]

    @pl.when(narrow)
    def _():
        pw_ref[...] = (eb[:, None] == lax.broadcasted_iota(jnp.int32, (B, _W), 1)
                       + lo).astype(jnp.float32)
        gbuf_ref[...] = _dot(pw_ref[...], qf2_ref[pl.ds(lo, _W), :])

    @pl.when(jnp.logical_not(narrow))
    def _():
        P = (eb[:, None] == lax.broadcasted_iota(jnp.int32, (B, G), 1)
             ).astype(jnp.float32)
        gbuf_ref[...] = _dot(P, qf2_ref[:G, :])

    oidx = jnp.clip(ord_ref[0, 0, :], -1, R - 2) + 1
    Po = (oidx[:, None] == lax.broadcasted_iota(jnp.int32, (B, R), 1)
          ).astype(jnp.float32)

    et = et_ref[...] + _dot(Po, ordemb_ref[...]) + gbuf_ref[...]
    x = _gelu(_dot(_lnn(et), m1W_ref[...]) + m1bias_ref[...])
    x = _gelu(_dot(_lnn(x), m2W_ref[...]) + m2bias_ref[...])
    x_ref[...] = x

    mf = mask_ref[0, 0, :].astype(jnp.float32)
    xm = x * mf[:, None]

    @pl.when(i == 0)
    def _():
        ssum_ref[...] = jnp.zeros_like(ssum_ref)
        cnt_ref[...] = jnp.zeros_like(cnt_ref)

    @pl.when(narrow)
    def _():
        Pw = pw_ref[...]
        ssum_ref[pl.ds(lo, _W), :] += lax.dot_general(
            Pw, xm, (((0,), (0,)), ((), ())), preferred_element_type=jnp.float32)
        cnt_ref[pl.ds(lo, _W), :] += lax.dot_general(
            Pw, mf[:, None], (((0,), (0,)), ((), ())),
            preferred_element_type=jnp.float32)

    @pl.when(jnp.logical_not(narrow))
    def _():
        P = (eb[:, None] == lax.broadcasted_iota(jnp.int32, (B, G), 1)
             ).astype(jnp.float32)
        ssum_ref[:G, :] += lax.dot_general(
            P, xm, (((0,), (0,)), ((), ())), preferred_element_type=jnp.float32)
        cnt_ref[:G, :] += lax.dot_general(
            P, mf[:, None], (((0,), (0,)), ((), ())),
            preferred_element_type=jnp.float32)


def _graph_kernel(ssum_ref, cnt_ref, q_ref, gng_ref, gnb_ref,
                  spW1_ref, spb1_ref, spW2_ref, spb2_ref,
                  cs_ref, stop_ref, tab_ref, r_ref):
    G = q_ref.shape[0]
    ssum = ssum_ref[:G, :]
    q = q_ref[...]
    cnt_raw = cnt_ref[:G, :]
    cnt_c = jnp.maximum(cnt_raw, 1.0)
    cs = _ln(ssum / cnt_c + q, gng_ref[...], gnb_ref[...])
    cs_ref[...] = cs
    r = 1.0 / (cnt_raw + 1.0)
    r_ref[:G, :] = r
    r_ref[G:, :] = jnp.zeros_like(r_ref[G:, :])
    tab_ref[:G, :] = jnp.concatenate([cs, ssum * r + q], axis=1)
    tab_ref[G:, :] = jnp.zeros_like(tab_ref[G:, :])
    catg = jnp.concatenate([cs, q], axis=1)
    hs = _gelu(_dot(_lnn(catg), spW1_ref[...]) + spb1_ref[...])
    stop_ref[...] = _dot(hs, spW2_ref[...]) + spb2_ref[...]


def _edge_head_kernel(lo_ref, hi_ref, x_ref, eb_ref, tab_ref, r_ref,
                      gng_ref, gnb_ref, lhW1_ref, lhb1_ref, lhW2_ref, lhb2_ref,
                      out_ref, gbuf_ref, rbuf_ref):
    i = pl.program_id(0)
    B = x_ref.shape[0]
    G = tab_ref.shape[0] - _W
    H = x_ref.shape[1]

    lo = (lo_ref[i] // 8) * 8
    narrow = hi_ref[i] - lo < _W
    eb = eb_ref[0, 0, :]

    @pl.when(narrow)
    def _():
        Pw = (eb[:, None] == lax.broadcasted_iota(jnp.int32, (B, _W), 1) + lo
              ).astype(jnp.float32)
        gbuf_ref[...] = _dot(Pw, tab_ref[pl.ds(lo, _W), :])
        rbuf_ref[...] = _dot(Pw, r_ref[pl.ds(lo, _W), :])

    @pl.when(jnp.logical_not(narrow))
    def _():
        P = (eb[:, None] == lax.broadcasted_iota(jnp.int32, (B, G), 1)
             ).astype(jnp.float32)
        gbuf_ref[...] = _dot(P, tab_ref[:G, :])
        rbuf_ref[...] = _dot(P, r_ref[:G, :])

    gath = gbuf_ref[...]
    cs_e = gath[:, :H]
    a_e = gath[:, H:]
    v = x_ref[...] * rbuf_ref[...] + a_e
    ns = _ln(v, gng_ref[...], gnb_ref[...])
    cat = jnp.concatenate([cs_e, ns], axis=1)
    h = _gelu(_dot(_lnn(cat), lhW1_ref[...]) + lhb1_ref[...])
    out_ref[...] = _dot(h, lhW2_ref[...]) + lhb2_ref[...]


def kernel(edge_tokens, question_tokens, edge_batch, selected_mask, selection_order,
           order_emb, type_emb, gn_g, gn_b, qf_g, qf_b, qf_W, qf_bias,
           m1_g, m1_b, m1_W, m1_bias, m2_g, m2_b, m2_W, m2_bias,
           lh_g, lh_b, lh_W1, lh_b1, lh_W2, lh_b2,
           sp_g, sp_b, sp_W1, sp_b1, sp_W2, sp_b2):
    E, H = edge_tokens.shape
    G = question_tokens.shape[0]
    B = _pick_block(E)
    NB = E // B
    GP = G + _W

    row = lambda a: a.reshape(1, -1)

    qf_Wf = qf_g[:, None] * qf_W
    qf_biasf = row(qf_bias + qf_b @ qf_W + type_emb[2])
    m1_Wf = m1_g[:, None] * m1_W
    m1_biasf = row(m1_bias + m1_b @ m1_W)
    m2_Wf = m2_g[:, None] * m2_W
    m2_biasf = row(m2_bias + m2_b @ m2_W)
    lh_W1f = lh_g[:, None] * lh_W1
    lh_b1f = row(lh_b1 + lh_b @ lh_W1)
    sp_W1f = sp_g[:, None] * sp_W1
    sp_b1f = row(sp_b1 + sp_b @ sp_W1)

    ordemb = jnp.concatenate(
        [order_emb, jnp.zeros((16 - order_emb.shape[0], H), order_emb.dtype)], axis=0)

    qf2 = pl.pallas_call(
        _qf_kernel,
        out_shape=jax.ShapeDtypeStruct((G, H), jnp.float32),
    )(question_tokens, qf_Wf, qf_biasf)
    qf2p = jnp.concatenate([qf2, jnp.zeros((_W, H), jnp.float32)], axis=0)

    ebi = edge_batch.astype(jnp.int32)
    los = ebi[0::B]
    his = ebi[B - 1::B]
    eb3 = ebi.reshape(NB, 1, B)
    ord3 = selection_order.astype(jnp.int32).reshape(NB, 1, B)
    mask3 = selected_mask.astype(jnp.int32).reshape(NB, 1, B)

    full = lambda shape: pl.BlockSpec(shape, lambda i: (0,) * len(shape))
    smem = pl.BlockSpec(memory_space=pltpu.SMEM)
    idx3 = pl.BlockSpec((1, 1, B), lambda i: (i, 0, 0))
    eblk = pl.BlockSpec((B, H), lambda i: (i, 0))

    x, ssum, cnt = pl.pallas_call(
        _edge_fwd_kernel,
        grid=(NB,),
        in_specs=[smem, smem, eblk, idx3, idx3, idx3, full((GP, H)), full((16, H)),
                  full((H, H)), full((1, H)), full((H, H)), full((1, H))],
        out_specs=[eblk, full((GP, H)), full((GP, 1))],
        out_shape=[jax.ShapeDtypeStruct((E, H), jnp.float32),
                   jax.ShapeDtypeStruct((GP, H), jnp.float32),
                   jax.ShapeDtypeStruct((GP, 1), jnp.float32)],
        scratch_shapes=[pltpu.VMEM((B, H), jnp.float32),
                        pltpu.VMEM((B, _W), jnp.float32)],
    )(los, his, edge_tokens, eb3, ord3, mask3, qf2p, ordemb,
      m1_Wf, m1_biasf, m2_Wf, m2_biasf)

    cs, stop, tab, r = pl.pallas_call(
        _graph_kernel,
        out_shape=[jax.ShapeDtypeStruct((G, H), jnp.float32),
                   jax.ShapeDtypeStruct((G, 1), jnp.float32),
                   jax.ShapeDtypeStruct((GP, 2 * H), jnp.float32),
                   jax.ShapeDtypeStruct((GP, 1), jnp.float32)],
    )(ssum, cnt, question_tokens, row(gn_g), row(gn_b),
      sp_W1f, sp_b1f, sp_W2, sp_b2.reshape(1, 1))

    logits = pl.pallas_call(
        _edge_head_kernel,
        grid=(NB,),
        in_specs=[smem, smem, eblk, idx3, full((GP, 2 * H)), full((GP, 1)),
                  full((1, H)), full((1, H)),
                  full((2 * H, H)), full((1, H)), full((H, 1)), full((1, 1))],
        out_specs=pl.BlockSpec((B, 1), lambda i: (i, 0)),
        out_shape=jax.ShapeDtypeStruct((E, 1), jnp.float32),
        scratch_shapes=[pltpu.VMEM((B, 2 * H), jnp.float32),
                        pltpu.VMEM((B, 1), jnp.float32)],
    )(los, his, x, eb3, tab, r, row(gn_g), row(gn_b),
      lh_W1f, lh_b1f, lh_W2, lh_b2.reshape(1, 1))

    return logits[:, 0], stop[:, 0], cs


# speedup vs baseline: 1.8483x; 1.0430x over previous
import jax
import jax.numpy as jnp
from jax import lax
from jax.experimental import pallas as pl
from jax.experimental.pallas import tpu as pltpu

_SQRT2 = 1.4142135623730951
_W = 128


def _gelu(x):
    return x * 0.5 * (1.0 + lax.erf(x / _SQRT2))


def _lnn(x):
    m = jnp.mean(x, axis=-1, keepdims=True)
    ms = jnp.mean(x * x, axis=-1, keepdims=True)
    rs = lax.rsqrt(jnp.maximum(ms - m * m, 0.0) + 1e-5)
    return x * rs - m * rs


def _ln(x, g, b):
    m = jnp.mean(x, axis=-1, keepdims=True)
    ms = jnp.mean(x * x, axis=-1, keepdims=True)
    rs = lax.rsqrt(jnp.maximum(ms - m * m, 0.0) + 1e-5)
    return (x * rs - m * rs) * g + b


def _dot(a, b):
    return jnp.dot(a, b, preferred_element_type=jnp.float32)


def _pick_block(e):
    for b in (4000, 2000, 1600, 1280, 1024, 1000, 800, 640, 512, 256, 128, 64, 32, 16, 8):
        if e % b == 0:
            return b
    return e


def _qf_kernel(q_ref, qfW_ref, qfbias_ref, out_ref):
    y = _dot(_lnn(q_ref[...]), qfW_ref[...]) + qfbias_ref[...]
    out_ref[...] = _gelu(y)


def _make_fwd_kernel(win, windowed):
    def body(lo_ref, et_ref, eb_ref, ord_ref, mask_ref,
             qf2_ref, ordemb_ref, m1W_ref, m1bias_ref, m2W_ref, m2bias_ref,
             x_ref, ssum_ref, cnt_ref):
        i = pl.program_id(0)
        B = et_ref.shape[0]
        R = ordemb_ref.shape[0]
        lo = (lo_ref[i] // 8) * 8 if windowed else 0

        eb = eb_ref[0, 0, :]
        Pw = (eb[:, None] == lax.broadcasted_iota(jnp.int32, (B, win), 1) + lo
              ).astype(jnp.float32)
        oidx = jnp.clip(ord_ref[0, 0, :], -1, R - 2) + 1
        Po = (oidx[:, None] == lax.broadcasted_iota(jnp.int32, (B, R), 1)
              ).astype(jnp.float32)

        et = (et_ref[...] + _dot(Po, ordemb_ref[...])
              + _dot(Pw, qf2_ref[pl.ds(lo, win), :]))
        x = _gelu(_dot(_lnn(et), m1W_ref[...]) + m1bias_ref[...])
        x = _gelu(_dot(_lnn(x), m2W_ref[...]) + m2bias_ref[...])
        x_ref[...] = x

        mf = mask_ref[0, 0, :].astype(jnp.float32)
        xm = x * mf[:, None]

        @pl.when(i == 0)
        def _():
            ssum_ref[...] = jnp.zeros_like(ssum_ref)
            cnt_ref[...] = jnp.zeros_like(cnt_ref)

        ssum_ref[pl.ds(lo, win), :] += lax.dot_general(
            Pw, xm, (((0,), (0,)), ((), ())), preferred_element_type=jnp.float32)
        cnt_ref[pl.ds(lo, win), :] += lax.dot_general(
            Pw, mf[:, None], (((0,), (0,)), ((), ())),
            preferred_element_type=jnp.float32)

    return body


def _graph_kernel(ssum_ref, cnt_ref, q_ref, gng_ref, gnb_ref,
                  spW1_ref, spb1_ref, spW2_ref, spb2_ref,
                  cs_ref, stop_ref, tab_ref, r_ref):
    G = q_ref.shape[0]
    ssum = ssum_ref[:G, :]
    q = q_ref[...]
    cnt_raw = cnt_ref[:G, :]
    cnt_c = jnp.maximum(cnt_raw, 1.0)
    cs = _ln(ssum / cnt_c + q, gng_ref[...], gnb_ref[...])
    cs_ref[...] = cs
    r = 1.0 / (cnt_raw + 1.0)
    r_ref[:G, :] = r
    r_ref[G:, :] = jnp.zeros_like(r_ref[G:, :])
    tab_ref[:G, :] = jnp.concatenate([cs, ssum * r + q], axis=1)
    tab_ref[G:, :] = jnp.zeros_like(tab_ref[G:, :])
    catg = jnp.concatenate([cs, q], axis=1)
    hs = _gelu(_dot(_lnn(catg), spW1_ref[...]) + spb1_ref[...])
    stop_ref[...] = _dot(hs, spW2_ref[...]) + spb2_ref[...]


def _make_head_kernel(win, windowed):
    def body(lo_ref, x_ref, eb_ref, tab_ref, r_ref,
             gng_ref, gnb_ref, lhW1_ref, lhb1_ref, lhW2_ref, lhb2_ref,
             out_ref):
        i = pl.program_id(0)
        B = x_ref.shape[0]
        H = x_ref.shape[1]
        lo = (lo_ref[i] // 8) * 8 if windowed else 0

        eb = eb_ref[0, 0, :]
        Pw = (eb[:, None] == lax.broadcasted_iota(jnp.int32, (B, win), 1) + lo
              ).astype(jnp.float32)
        gath = _dot(Pw, tab_ref[pl.ds(lo, win), :])
        r_e = _dot(Pw, r_ref[pl.ds(lo, win), :])
        cs_e = gath[:, :H]
        a_e = gath[:, H:]
        v = x_ref[...] * r_e + a_e
        ns = _ln(v, gng_ref[...], gnb_ref[...])
        cat = jnp.concatenate([cs_e, ns], axis=1)
        h = _gelu(_dot(_lnn(cat), lhW1_ref[...]) + lhb1_ref[...])
        out_ref[...] = _dot(h, lhW2_ref[...]) + lhb2_ref[...]

    return body


def kernel(edge_tokens, question_tokens, edge_batch, selected_mask, selection_order,
           order_emb, type_emb, gn_g, gn_b, qf_g, qf_b, qf_W, qf_bias,
           m1_g, m1_b, m1_W, m1_bias, m2_g, m2_b, m2_W, m2_bias,
           lh_g, lh_b, lh_W1, lh_b1, lh_W2, lh_b2,
           sp_g, sp_b, sp_W1, sp_b1, sp_W2, sp_b2):
    E, H = edge_tokens.shape
    G = question_tokens.shape[0]
    B = _pick_block(E)
    NB = E // B
    GP = G + _W

    row = lambda a: a.reshape(1, -1)

    qf_Wf = qf_g[:, None] * qf_W
    qf_biasf = row(qf_bias + qf_b @ qf_W + type_emb[2])
    m1_Wf = m1_g[:, None] * m1_W
    m1_biasf = row(m1_bias + m1_b @ m1_W)
    m2_Wf = m2_g[:, None] * m2_W
    m2_biasf = row(m2_bias + m2_b @ m2_W)
    lh_W1f = lh_g[:, None] * lh_W1
    lh_b1f = row(lh_b1 + lh_b @ lh_W1)
    sp_W1f = sp_g[:, None] * sp_W1
    sp_b1f = row(sp_b1 + sp_b @ sp_W1)

    ordemb = jnp.concatenate(
        [order_emb, jnp.zeros((16 - order_emb.shape[0], H), order_emb.dtype)], axis=0)

    qf2 = pl.pallas_call(
        _qf_kernel,
        out_shape=jax.ShapeDtypeStruct((G, H), jnp.float32),
    )(question_tokens, qf_Wf, qf_biasf)
    qf2p = jnp.concatenate([qf2, jnp.zeros((_W, H), jnp.float32)], axis=0)

    ebi = edge_batch.astype(jnp.int32)
    los = ebi[0::B]
    narrow_ok = jnp.max(ebi[B - 1::B] - (los // 8) * 8) < _W
    eb3 = ebi.reshape(NB, 1, B)
    ord3 = selection_order.astype(jnp.int32).reshape(NB, 1, B)
    mask3 = selected_mask.astype(jnp.int32).reshape(NB, 1, B)

    full = lambda shape: pl.BlockSpec(shape, lambda i: (0,) * len(shape))
    smem = pl.BlockSpec(memory_space=pltpu.SMEM)
    idx3 = pl.BlockSpec((1, 1, B), lambda i: (i, 0, 0))
    eblk = pl.BlockSpec((B, H), lambda i: (i, 0))

    def fwd(win, windowed):
        return pl.pallas_call(
            _make_fwd_kernel(win, windowed),
            grid=(NB,),
            in_specs=[smem, eblk, idx3, idx3, idx3, full((GP, H)), full((16, H)),
                      full((H, H)), full((1, H)), full((H, H)), full((1, H))],
            out_specs=[eblk, full((GP, H)), full((GP, 1))],
            out_shape=[jax.ShapeDtypeStruct((E, H), jnp.float32),
                       jax.ShapeDtypeStruct((GP, H), jnp.float32),
                       jax.ShapeDtypeStruct((GP, 1), jnp.float32)],
        )(los, edge_tokens, eb3, ord3, mask3, qf2p, ordemb,
          m1_Wf, m1_biasf, m2_Wf, m2_biasf)

    x, ssum, cnt = lax.cond(
        narrow_ok,
        lambda: fwd(_W, True),
        lambda: fwd(G, False))

    cs, stop, tab, r = pl.pallas_call(
        _graph_kernel,
        out_shape=[jax.ShapeDtypeStruct((G, H), jnp.float32),
                   jax.ShapeDtypeStruct((G, 1), jnp.float32),
                   jax.ShapeDtypeStruct((GP, 2 * H), jnp.float32),
                   jax.ShapeDtypeStruct((GP, 1), jnp.float32)],
    )(ssum, cnt, question_tokens, row(gn_g), row(gn_b),
      sp_W1f, sp_b1f, sp_W2, sp_b2.reshape(1, 1))

    def head(win, windowed):
        return pl.pallas_call(
            _make_head_kernel(win, windowed),
            grid=(NB,),
            in_specs=[smem, eblk, idx3, full((GP, 2 * H)), full((GP, 1)),
                      full((1, H)), full((1, H)),
                      full((2 * H, H)), full((1, H)), full((H, 1)), full((1, 1))],
            out_specs=pl.BlockSpec((B, 1), lambda i: (i, 0)),
            out_shape=jax.ShapeDtypeStruct((E, 1), jnp.float32),
        )(los, x, eb3, tab, r, row(gn_g), row(gn_b),
          lh_W1f, lh_b1f, lh_W2, lh_b2.reshape(1, 1))

    logits = lax.cond(
        narrow_ok,
        lambda: head(_W, True),
        lambda: head(G, False))

    return logits[:, 0], stop[:, 0], cs
